# baseline (speedup 1.0000x reference)
import functools

import jax
import jax.numpy as jnp
from jax import lax
from jax.experimental import pallas as pl
from jax.experimental.pallas import tpu as pltpu
from jax.experimental.pallas import tpu_sc as plsc

_DIM = 768
_LANES = 16
_NCHUNK = _DIM // _LANES
_NCORES = 2
_NSUB = 16
_NWORKERS = _NCORES * _NSUB
_EPS = 1e-12


def _rsqrt(x):
    i = lax.bitcast_convert_type(x, jnp.int32)
    i = jnp.int32(0x5F3759DF) - lax.shift_right_logical(i, 1)
    y = lax.bitcast_convert_type(i, jnp.float32)
    h = x * 0.5
    y = y * (1.5 - h * y * y)
    y = y * (1.5 - h * y * y)
    y = y * (1.5 - h * y * y)
    return y


def _lane_sum(v):
    parts = [v[i] for i in range(_LANES)]
    while len(parts) > 1:
        parts = [parts[i] + parts[i + 1] for i in range(0, len(parts), 2)]
    return parts[0]


def _sc_body(ntok_per_w, k_tok, xf_hbm, cidx_hbm, table_hbm, comb_hbm, gb_hbm,
             out_hbm, idx_v, cidx_v, gb_v, tok_v, comb_v, sem):
    wid = lax.axis_index("s") * _NCORES + lax.axis_index("c")
    base = wid * ntok_per_w
    pltpu.sync_copy(xf_hbm.at[pl.ds(base, ntok_per_w)], idx_v)
    pltpu.sync_copy(cidx_hbm.at[pl.ds(base, ntok_per_w)], cidx_v)
    pltpu.sync_copy(gb_hbm, gb_v)

    n_steps = ntok_per_w // k_tok

    def step_fn(step, carry):
        off = step * k_tok
        pltpu.async_copy(table_hbm.at[idx_v.at[pl.ds(off, k_tok)]], tok_v, sem).wait()
        pltpu.async_copy(comb_hbm.at[cidx_v.at[pl.ds(off, k_tok)]], comb_v, sem).wait()

        def tok_fn(j, carry2):
            s_acc = [jnp.zeros((_LANES,), jnp.float32) for _ in range(4)]
            q_acc = [jnp.zeros((_LANES,), jnp.float32) for _ in range(4)]
            for c in range(_NCHUNK):
                v = tok_v[j, pl.ds(c * _LANES, _LANES)] + comb_v[j, pl.ds(c * _LANES, _LANES)]
                tok_v[j, pl.ds(c * _LANES, _LANES)] = v
                s_acc[c % 4] = s_acc[c % 4] + v
                q_acc[c % 4] = q_acc[c % 4] + v * v
            s_vec = (s_acc[0] + s_acc[1]) + (s_acc[2] + s_acc[3])
            q_vec = (q_acc[0] + q_acc[1]) + (q_acc[2] + q_acc[3])
            tot = _lane_sum(s_vec)
            qtot = _lane_sum(q_vec)
            u = tot * (1.0 / _DIM)
            var = qtot * (1.0 / _DIM) - u * u
            r = _rsqrt(var + _EPS)
            a = -u * r
            for c in range(_NCHUNK):
                v = tok_v[j, pl.ds(c * _LANES, _LANES)]
                tok_v[j, pl.ds(c * _LANES, _LANES)] = v * r + (gb_v[pl.ds(c * _LANES, _LANES)] + a)
            return carry2

        lax.fori_loop(0, k_tok, tok_fn, 0)
        pltpu.sync_copy(tok_v, out_hbm.at[pl.ds(base + off, k_tok)])
        return carry

    lax.fori_loop(0, n_steps, step_fn, 0)


def kernel(x, segment, token_table, pos_table, seg_table, gamma, beta):
    bsz, seq_len = x.shape
    n_tok = bsz * seq_len
    ntok_per_w = n_tok // _NWORKERS
    k_tok = 32

    xf = x.reshape(-1).astype(jnp.int32)
    pos_ids = jnp.arange(seq_len, dtype=jnp.int32)
    cidx = (segment.astype(jnp.int32) * seq_len + pos_ids[None, :]).reshape(-1)
    comb = (seg_table[:, None, :] + pos_table[None, :seq_len, :]).reshape(-1, _DIM)
    gb = gamma + beta

    mesh = plsc.VectorSubcoreMesh(
        core_axis_name="c", subcore_axis_name="s",
        num_cores=_NCORES, num_subcores=_NSUB)

    body = functools.partial(_sc_body, ntok_per_w, k_tok)
    out = pl.kernel(
        body,
        out_type=jax.ShapeDtypeStruct((n_tok, _DIM), jnp.float32),
        mesh=mesh,
        scratch_types=[
            pltpu.VMEM((ntok_per_w,), jnp.int32),
            pltpu.VMEM((ntok_per_w,), jnp.int32),
            pltpu.VMEM((_DIM,), jnp.float32),
            pltpu.VMEM((k_tok, _DIM), jnp.float32),
            pltpu.VMEM((k_tok, _DIM), jnp.float32),
            pltpu.SemaphoreType.DMA,
        ],
    )(xf, cidx, token_table, comb, gb)
    return out.reshape(bsz, seq_len, _DIM)


# speedup vs baseline: 1.2710x; 1.2710x over previous
import functools

import jax
import jax.numpy as jnp
from jax import lax
from jax.experimental import pallas as pl
from jax.experimental.pallas import tpu as pltpu
from jax.experimental.pallas import tpu_sc as plsc

_DIM = 768
_LANES = 16
_NCHUNK = _DIM // _LANES
_NCORES = 2
_NSUB = 16
_NWORKERS = _NCORES * _NSUB
_NBUF = 4
_EPS = 1e-12


def _rsqrt(x):
    i = lax.bitcast_convert_type(x, jnp.int32)
    i = jnp.int32(0x5F3759DF) - lax.shift_right_logical(i, 1)
    y = lax.bitcast_convert_type(i, jnp.float32)
    h = x * 0.5
    y = y * (1.5 - h * y * y)
    y = y * (1.5 - h * y * y)
    y = y * (1.5 - h * y * y)
    return y


def _lane_sum(v):
    parts = [v[i] for i in range(_LANES)]
    while len(parts) > 1:
        parts = [parts[i] + parts[i + 1] for i in range(0, len(parts), 2)]
    return parts[0]


def _normalize_block(tok_v, comb_v, gb_v, k_tok):
    def tok_fn(j, carry):
        s_acc = [jnp.zeros((_LANES,), jnp.float32) for _ in range(4)]
        q_acc = [jnp.zeros((_LANES,), jnp.float32) for _ in range(4)]
        for c in range(_NCHUNK):
            v = tok_v[j, pl.ds(c * _LANES, _LANES)] + comb_v[j, pl.ds(c * _LANES, _LANES)]
            tok_v[j, pl.ds(c * _LANES, _LANES)] = v
            s_acc[c % 4] = s_acc[c % 4] + v
            q_acc[c % 4] = q_acc[c % 4] + v * v
        s_vec = (s_acc[0] + s_acc[1]) + (s_acc[2] + s_acc[3])
        q_vec = (q_acc[0] + q_acc[1]) + (q_acc[2] + q_acc[3])
        tot = _lane_sum(s_vec)
        qtot = _lane_sum(q_vec)
        u = tot * (1.0 / _DIM)
        var = qtot * (1.0 / _DIM) - u * u
        r = _rsqrt(var + _EPS)
        a = -u * r
        for c in range(_NCHUNK):
            v = tok_v[j, pl.ds(c * _LANES, _LANES)]
            tok_v[j, pl.ds(c * _LANES, _LANES)] = v * r + (gb_v[pl.ds(c * _LANES, _LANES)] + a)
        return carry

    lax.fori_loop(0, k_tok, tok_fn, 0)


def _sc_body(ntok_per_w, k_tok, xf_hbm, cidx_hbm, table_hbm, comb_hbm, gb_hbm,
             out_hbm, idx_v, cidx_v, gb_v, tok_bufs, comb_bufs, gsems, osems):
    wid = lax.axis_index("s") * _NCORES + lax.axis_index("c")
    base = wid * ntok_per_w
    pltpu.sync_copy(xf_hbm.at[pl.ds(base, ntok_per_w)], idx_v)
    pltpu.sync_copy(cidx_hbm.at[pl.ds(base, ntok_per_w)], cidx_v)
    pltpu.sync_copy(gb_hbm, gb_v)

    n_steps = ntok_per_w // k_tok

    def start_gather(step, b):
        off = step * k_tok
        pltpu.async_copy(table_hbm.at[idx_v.at[pl.ds(off, k_tok)]],
                         tok_bufs[b], gsems[b])
        pltpu.async_copy(comb_hbm.at[cidx_v.at[pl.ds(off, k_tok)]],
                         comb_bufs[b], gsems[b])

    def wait_gather(b):
        pltpu.make_async_copy(table_hbm.at[idx_v.at[pl.ds(0, k_tok)]],
                              tok_bufs[b], gsems[b]).wait()
        pltpu.make_async_copy(comb_hbm.at[cidx_v.at[pl.ds(0, k_tok)]],
                              comb_bufs[b], gsems[b]).wait()

    def start_out(step, b):
        pltpu.async_copy(tok_bufs[b], out_hbm.at[pl.ds(base + step * k_tok, k_tok)],
                         osems[b])

    def wait_out(b):
        pltpu.make_async_copy(tok_bufs[b], out_hbm.at[pl.ds(base, k_tok)],
                              osems[b]).wait()

    start_gather(0, 0)
    start_gather(1, 1)

    def quad_fn(q, carry):
        for i in range(_NBUF):
            step = q * _NBUF + i
            z = (i + 2) % _NBUF

            @pl.when(step >= 2)
            def _():
                wait_out(z)

            @pl.when(step + 2 < n_steps)
            def _():
                start_gather(step + 2, z)

            wait_gather(i)
            _normalize_block(tok_bufs[i], comb_bufs[i], gb_v, k_tok)
            start_out(step, i)
        return carry

    lax.fori_loop(0, n_steps // _NBUF, quad_fn, 0)
    wait_out((n_steps - 2) % _NBUF)
    wait_out((n_steps - 1) % _NBUF)


def kernel(x, segment, token_table, pos_table, seg_table, gamma, beta):
    bsz, seq_len = x.shape
    n_tok = bsz * seq_len
    ntok_per_w = n_tok // _NWORKERS
    k_tok = 16

    xf = x.reshape(-1).astype(jnp.int32)
    pos_ids = jnp.arange(seq_len, dtype=jnp.int32)
    cidx = (segment.astype(jnp.int32) * seq_len + pos_ids[None, :]).reshape(-1)
    comb = (seg_table[:, None, :] + pos_table[None, :seq_len, :]).reshape(-1, _DIM)
    gb = gamma + beta

    mesh = plsc.VectorSubcoreMesh(
        core_axis_name="c", subcore_axis_name="s",
        num_cores=_NCORES, num_subcores=_NSUB)

    body = functools.partial(_sc_body, ntok_per_w, k_tok)
    out = pl.kernel(
        body,
        out_type=jax.ShapeDtypeStruct((n_tok, _DIM), jnp.float32),
        mesh=mesh,
        scratch_types=[
            pltpu.VMEM((ntok_per_w,), jnp.int32),
            pltpu.VMEM((ntok_per_w,), jnp.int32),
            pltpu.VMEM((_DIM,), jnp.float32),
            [pltpu.VMEM((k_tok, _DIM), jnp.float32) for _ in range(_NBUF)],
            [pltpu.VMEM((k_tok, _DIM), jnp.float32) for _ in range(_NBUF)],
            [pltpu.SemaphoreType.DMA for _ in range(_NBUF)],
            [pltpu.SemaphoreType.DMA for _ in range(_NBUF)],
        ],
    )(xf, cidx, token_table, comb, gb)
    return out.reshape(bsz, seq_len, _DIM)


# speedup vs baseline: 1.4883x; 1.1710x over previous
import functools

import jax
import jax.numpy as jnp
from jax import lax
from jax.experimental import pallas as pl
from jax.experimental.pallas import tpu as pltpu
from jax.experimental.pallas import tpu_sc as plsc

_DIM = 768
_LANES = 16
_NCHUNK = _DIM // _LANES
_NCORES = 2
_NSUB = 16
_NWORKERS = _NCORES * _NSUB
_NBUF = 4
_EPS = 1e-12


def _rsqrt(x):
    i = lax.bitcast_convert_type(x, jnp.int32)
    i = jnp.int32(0x5F3759DF) - lax.shift_right_logical(i, 1)
    y = lax.bitcast_convert_type(i, jnp.float32)
    h = x * 0.5
    y = y * (1.5 - h * y * y)
    y = y * (1.5 - h * y * y)
    y = y * (1.5 - h * y * y)
    return y


def _lane_sum(v):
    parts = [v[i] for i in range(_LANES)]
    while len(parts) > 1:
        parts = [parts[i] + parts[i + 1] for i in range(0, len(parts), 2)]
    return parts[0]


def _normalize_block(tok_v, comb_v, gb_v, k_tok):
    def pass1(j):
        s_acc = [jnp.zeros((_LANES,), jnp.float32) for _ in range(4)]
        q_acc = [jnp.zeros((_LANES,), jnp.float32) for _ in range(4)]
        for c in range(_NCHUNK):
            v = tok_v[j, pl.ds(c * _LANES, _LANES)] + comb_v[j, pl.ds(c * _LANES, _LANES)]
            tok_v[j, pl.ds(c * _LANES, _LANES)] = v
            s_acc[c % 4] = s_acc[c % 4] + v
            q_acc[c % 4] = q_acc[c % 4] + v * v
        s_vec = (s_acc[0] + s_acc[1]) + (s_acc[2] + s_acc[3])
        q_vec = (q_acc[0] + q_acc[1]) + (q_acc[2] + q_acc[3])
        return s_vec, q_vec

    def reduce_pass2(j, s_vec, q_vec):
        tot = _lane_sum(s_vec)
        qtot = _lane_sum(q_vec)
        u = tot * (1.0 / _DIM)
        var = qtot * (1.0 / _DIM) - u * u
        r = _rsqrt(var + _EPS)
        a = -u * r
        for c in range(_NCHUNK):
            v = tok_v[j, pl.ds(c * _LANES, _LANES)]
            tok_v[j, pl.ds(c * _LANES, _LANES)] = v * r + (gb_v[pl.ds(c * _LANES, _LANES)] + a)

    def loop_fn(j, carry):
        s_vec, q_vec = carry
        nxt = pass1(j + 1)
        reduce_pass2(j, s_vec, q_vec)
        return nxt

    carry = pass1(0)
    carry = lax.fori_loop(0, k_tok - 1, loop_fn, carry)
    reduce_pass2(k_tok - 1, *carry)


def _sc_body(ntok_per_w, k_tok, xf_hbm, cidx_hbm, table_hbm, comb_hbm, gb_hbm,
             out_hbm, idx_v, cidx_v, gb_v, tok_bufs, comb_bufs,
             gsems, osems):
    wid = lax.axis_index("s") * _NCORES + lax.axis_index("c")
    base = wid * ntok_per_w
    pltpu.sync_copy(xf_hbm.at[pl.ds(base, ntok_per_w)], idx_v)
    pltpu.sync_copy(cidx_hbm.at[pl.ds(base, ntok_per_w)], cidx_v)
    pltpu.sync_copy(gb_hbm, gb_v)

    n_steps = ntok_per_w // k_tok

    def start_gather(step, b):
        off = step * k_tok
        pltpu.async_copy(table_hbm.at[idx_v.at[pl.ds(off, k_tok)]],
                         tok_bufs[b], gsems[b])
        pltpu.async_copy(comb_hbm.at[cidx_v.at[pl.ds(off, k_tok)]],
                         comb_bufs[b], gsems[b])

    def wait_gather(b):
        pltpu.make_async_copy(table_hbm.at[idx_v.at[pl.ds(0, k_tok)]],
                              tok_bufs[b], gsems[b]).wait()
        pltpu.make_async_copy(comb_hbm.at[cidx_v.at[pl.ds(0, k_tok)]],
                              comb_bufs[b], gsems[b]).wait()

    def start_out(step, b):
        pltpu.async_copy(tok_bufs[b], out_hbm.at[pl.ds(base + step * k_tok, k_tok)],
                         osems[b])

    def wait_out(b):
        pltpu.make_async_copy(tok_bufs[b], out_hbm.at[pl.ds(base, k_tok)],
                              osems[b]).wait()

    start_gather(0, 0)
    start_gather(1, 1)

    def quad_fn(q, carry):
        for i in range(_NBUF):
            step = q * _NBUF + i
            z = (i + 2) % _NBUF

            @pl.when(step >= 2)
            def _():
                wait_out(z)

            @pl.when(step + 2 < n_steps)
            def _():
                start_gather(step + 2, z)

            wait_gather(i)
            _normalize_block(tok_bufs[i], comb_bufs[i], gb_v, k_tok)
            start_out(step, i)
        return carry

    lax.fori_loop(0, n_steps // _NBUF, quad_fn, 0)
    wait_out((n_steps - 2) % _NBUF)
    wait_out((n_steps - 1) % _NBUF)


def kernel(x, segment, token_table, pos_table, seg_table, gamma, beta):
    bsz, seq_len = x.shape
    n_tok = bsz * seq_len
    ntok_per_w = n_tok // _NWORKERS
    k_tok = 16

    xf = x.reshape(-1).astype(jnp.int32)
    pos_ids = jnp.arange(seq_len, dtype=jnp.int32)
    cidx = (segment.astype(jnp.int32) * seq_len + pos_ids[None, :]).reshape(-1)
    comb = (seg_table[:, None, :] + pos_table[None, :seq_len, :]).reshape(-1, _DIM)
    gb = gamma + beta

    mesh = plsc.VectorSubcoreMesh(
        core_axis_name="c", subcore_axis_name="s",
        num_cores=_NCORES, num_subcores=_NSUB)

    body = functools.partial(_sc_body, ntok_per_w, k_tok)
    out = pl.kernel(
        body,
        out_type=jax.ShapeDtypeStruct((n_tok, _DIM), jnp.float32),
        mesh=mesh,
        scratch_types=[
            pltpu.VMEM((ntok_per_w,), jnp.int32),
            pltpu.VMEM((ntok_per_w,), jnp.int32),
            pltpu.VMEM((_DIM,), jnp.float32),
            [pltpu.VMEM((k_tok, _DIM), jnp.float32) for _ in range(_NBUF)],
            [pltpu.VMEM((k_tok, _DIM), jnp.float32) for _ in range(_NBUF)],
            [pltpu.SemaphoreType.DMA for _ in range(_NBUF)],
            [pltpu.SemaphoreType.DMA for _ in range(_NBUF)],
        ],
    )(xf, cidx, token_table, comb, gb)
    return out.reshape(bsz, seq_len, _DIM)


# speedup vs baseline: 4.4852x; 3.0135x over previous
import functools

import jax
import jax.numpy as jnp
from jax import lax
from jax.experimental import pallas as pl
from jax.experimental.pallas import tpu as pltpu
from jax.experimental.pallas import tpu_sc as plsc

_DIM = 768
_LANES = 16
_NCHUNK = _DIM // _LANES
_NCORES = 2
_NSUB = 16
_NWORKERS = _NCORES * _NSUB
_NBUF = 4
_EPS = 1e-12


def _pack_bf16_words(t):
    n = t.shape[0]
    u = lax.bitcast_convert_type(t.astype(jnp.bfloat16), jnp.uint16).astype(jnp.uint32)
    r = u.reshape(n, _DIM // (2 * _LANES), 2, _LANES)
    words = r[:, :, 0, :] | (r[:, :, 1, :] << 16)
    return lax.bitcast_convert_type(words, jnp.int32).reshape(n, _DIM // 2)


def _rsqrt(x):
    i = lax.bitcast_convert_type(x, jnp.int32)
    i = jnp.int32(0x5F3759DF) - lax.shift_right_logical(i, 1)
    y = lax.bitcast_convert_type(i, jnp.float32)
    h = x * 0.5
    y = y * (1.5 - h * y * y)
    y = y * (1.5 - h * y * y)
    y = y * (1.5 - h * y * y)
    return y


def _lane_sum(v):
    folded = v + lax.rev(v, (0,))
    parts = [folded[i] for i in range(_LANES // 2)]
    while len(parts) > 1:
        parts = [parts[i] + parts[i + 1] for i in range(0, len(parts), 2)]
    return parts[0]


def _unpack_bf16_pair(w):
    lo = lax.bitcast_convert_type(lax.shift_left(w, 16), jnp.float32)
    hi = lax.bitcast_convert_type(w, jnp.float32)
    return lo, hi


def _normalize_block(tok_v, comb_v, gb_v, emb_v, k_tok):
    n_grp = _NCHUNK // 2
    _LA = 2

    def pass1(j):
        s_acc = [jnp.zeros((_LANES,), jnp.float32) for _ in range(4)]
        q_acc = [jnp.zeros((_LANES,), jnp.float32) for _ in range(4)]

        def load1(g):
            return (comb_v[j, pl.ds(g * _LANES, _LANES)],
                    tok_v[j, pl.ds(2 * g * _LANES, _LANES)],
                    tok_v[j, pl.ds((2 * g + 1) * _LANES, _LANES)])

        queue = [load1(g) for g in range(_LA)]
        for g in range(n_grp):
            if g + _LA < n_grp:
                queue.append(load1(g + _LA))
            w, t0, t1 = queue.pop(0)
            clo, chi = _unpack_bf16_pair(w)
            v0 = t0 + clo
            v1 = t1 + chi
            emb_v[j, pl.ds(2 * g * _LANES, _LANES)] = v0
            emb_v[j, pl.ds((2 * g + 1) * _LANES, _LANES)] = v1
            s_acc[g % 2] = s_acc[g % 2] + v0
            s_acc[2 + g % 2] = s_acc[2 + g % 2] + v1
            q_acc[g % 2] = q_acc[g % 2] + v0 * v0
            q_acc[2 + g % 2] = q_acc[2 + g % 2] + v1 * v1
        s_vec = (s_acc[0] + s_acc[1]) + (s_acc[2] + s_acc[3])
        q_vec = (q_acc[0] + q_acc[1]) + (q_acc[2] + q_acc[3])
        return s_vec, q_vec

    def reduce_pass2(j, s_vec, q_vec):
        tot = _lane_sum(s_vec)
        qtot = _lane_sum(q_vec)
        u = tot * (1.0 / _DIM)
        var = qtot * (1.0 / _DIM) - u * u
        r = _rsqrt(var + _EPS)
        a = -u * r

        def load2(g):
            return (gb_v[pl.ds(g * _LANES, _LANES)],
                    emb_v[j, pl.ds(2 * g * _LANES, _LANES)],
                    emb_v[j, pl.ds((2 * g + 1) * _LANES, _LANES)])

        queue = [load2(g) for g in range(_LA)]
        for g in range(n_grp):
            if g + _LA < n_grp:
                queue.append(load2(g + _LA))
            w, e0, e1 = queue.pop(0)
            glo, ghi = _unpack_bf16_pair(w)
            tok_v[j, pl.ds(2 * g * _LANES, _LANES)] = e0 * r + (glo + a)
            tok_v[j, pl.ds((2 * g + 1) * _LANES, _LANES)] = e1 * r + (ghi + a)

    @plsc.parallel_loop(0, k_tok, unroll=2)
    def _(j):
        s_vec, q_vec = pass1(j)
        reduce_pass2(j, s_vec, q_vec)


def _sc_body(ntok_per_w, k_tok, xf_hbm, cidx_hbm, table_hbm, comb_hbm, gb_hbm,
             out_hbm, idx_v, cidx_v, gb_v, emb_v, tok_bufs, comb_bufs,
             gsems, osems):
    wid = lax.axis_index("s") * _NCORES + lax.axis_index("c")
    base = wid * ntok_per_w
    pltpu.sync_copy(xf_hbm.at[pl.ds(base, ntok_per_w)], idx_v)
    pltpu.sync_copy(cidx_hbm.at[pl.ds(base, ntok_per_w)], cidx_v)
    pltpu.sync_copy(gb_hbm, gb_v)

    n_steps = ntok_per_w // k_tok

    def start_gather(step, b):
        off = step * k_tok
        pltpu.async_copy(table_hbm.at[idx_v.at[pl.ds(off, k_tok)]],
                         tok_bufs[b], gsems[b])
        pltpu.async_copy(comb_hbm.at[cidx_v.at[pl.ds(off, k_tok)]],
                         comb_bufs[b], gsems[b])

    def wait_gather(b):
        pltpu.make_async_copy(table_hbm.at[idx_v.at[pl.ds(0, k_tok)]],
                              tok_bufs[b], gsems[b]).wait()
        pltpu.make_async_copy(comb_hbm.at[cidx_v.at[pl.ds(0, k_tok)]],
                              comb_bufs[b], gsems[b]).wait()

    def start_out(step, b):
        pltpu.async_copy(tok_bufs[b], out_hbm.at[pl.ds(base + step * k_tok, k_tok)],
                         osems[b])

    def wait_out(b):
        pltpu.make_async_copy(tok_bufs[b], out_hbm.at[pl.ds(base, k_tok)],
                              osems[b]).wait()

    start_gather(0, 0)
    start_gather(1, 1)

    def quad_fn(q, carry):
        for i in range(_NBUF):
            step = q * _NBUF + i
            z = (i + 2) % _NBUF

            @pl.when(step >= 2)
            def _():
                wait_out(z)

            @pl.when(step + 2 < n_steps)
            def _():
                start_gather(step + 2, z)

            wait_gather(i)
            _normalize_block(tok_bufs[i], comb_bufs[i], gb_v, emb_v, k_tok)
            start_out(step, i)
        return carry

    lax.fori_loop(0, n_steps // _NBUF, quad_fn, 0)
    wait_out((n_steps - 2) % _NBUF)
    wait_out((n_steps - 1) % _NBUF)


def kernel(x, segment, token_table, pos_table, seg_table, gamma, beta):
    bsz, seq_len = x.shape
    n_tok = bsz * seq_len
    ntok_per_w = n_tok // _NWORKERS
    k_tok = 16

    xf = x.reshape(-1).astype(jnp.int32)
    pos_ids = jnp.arange(seq_len, dtype=jnp.int32)
    cidx = (segment.astype(jnp.int32) * seq_len + pos_ids[None, :]).reshape(-1)
    comb = (seg_table[:, None, :] + pos_table[None, :seq_len, :]).reshape(-1, _DIM)
    comb_w = _pack_bf16_words(comb)
    gb_w = _pack_bf16_words((gamma + beta)[None, :])[0]

    mesh = plsc.VectorSubcoreMesh(
        core_axis_name="c", subcore_axis_name="s",
        num_cores=_NCORES, num_subcores=_NSUB)

    body = functools.partial(_sc_body, ntok_per_w, k_tok)
    out = pl.kernel(
        body,
        out_type=jax.ShapeDtypeStruct((n_tok, _DIM), jnp.float32),
        mesh=mesh,
        scratch_types=[
            pltpu.VMEM((ntok_per_w,), jnp.int32),
            pltpu.VMEM((ntok_per_w,), jnp.int32),
            pltpu.VMEM((_DIM // 2,), jnp.int32),
            pltpu.VMEM((k_tok, _DIM), jnp.float32),
            [pltpu.VMEM((k_tok, _DIM), jnp.float32) for _ in range(_NBUF)],
            [pltpu.VMEM((k_tok, _DIM // 2), jnp.int32) for _ in range(_NBUF)],
            [pltpu.SemaphoreType.DMA for _ in range(_NBUF)],
            [pltpu.SemaphoreType.DMA for _ in range(_NBUF)],
        ],
    )(xf, cidx, token_table, comb_w, gb_w)
    return out.reshape(bsz, seq_len, _DIM)
